# baseline (speedup 1.0000x reference)
import dataclasses

import jax
import jax.numpy as jnp
import numpy as np
from jax import lax
from jax.experimental import pallas as pl
from jax.experimental.pallas import tpu as pltpu
from jax.experimental.pallas import tpu_sc as plsc

NUM_EMB = 1000000
DIM = 64
K = 2
BINS = 99999
MOD = 2147483647
A0, B0 = 976369, 1014797
A1, B1 = 1982627, 180523
QMAX = (MOD - 1) // BINS

NC, NS, LANES = 2, 16, 16
NW = NC * NS

CH = DIM + K
N_TOK, W_TOK = 4096, 200
TOKENS = N_TOK * W_TOK
PER_TILE = TOKENS // NW
C = 256
CHUNKS = PER_TILE // C
GSUB = 128


def _i32(v):
    return jnp.int32(v)


def _loop(n, step, body):
    def bf(i, carry):
        body(i * _i32(step))
        return carry

    lax.fori_loop(_i32(0), _i32(n // step), bf, _i32(0))


def _hash_mod(xv, a, b):
    x0 = xv & _i32(1023)
    x1 = lax.shift_right_logical(xv, _i32(10))
    t_lo = x0 * _i32(a) + _i32(b)
    t_hi = x1 * _i32(a)
    u = lax.shift_right_logical(t_hi, _i32(21))
    v = t_hi & _i32((1 << 21) - 1)
    r1 = lax.shift_left(v, _i32(10)) + u
    s = r1 + t_lo
    s = (s & _i32(0x7FFFFFFF)) + lax.shift_right_logical(s, _i32(31))
    s = jnp.where(s >= _i32(MOD), s - _i32(MOD), s)
    q = (s.astype(jnp.float32) * jnp.float32(1.0 / BINS)).astype(jnp.int32)
    q = jnp.clip(q, _i32(0), _i32(QMAX))
    r = s - q * _i32(BINS)
    r = jnp.where(r < _i32(0), r + _i32(BINS), r)
    r = jnp.where(r >= _i32(BINS), r - _i32(BINS), r)
    return r


def _sc_call(idx, emb, p16):
    mesh = plsc.VectorSubcoreMesh(
        core_axis_name="c", subcore_axis_name="s", num_cores=NC, num_subcores=NS
    )

    cp = pltpu.CompilerParams()
    if "needs_layout_passes" in pltpu.CompilerParams.__dataclass_fields__:
        cp = dataclasses.replace(cp, needs_layout_passes=False)
    if "use_tc_tiling_on_sc" in pltpu.CompilerParams.__dataclass_fields__:
        cp = dataclasses.replace(cp, use_tc_tiling_on_sc=False)

    CP = C + 1

    @pl.kernel(
        out_type=jax.ShapeDtypeStruct((CH, TOKENS), jnp.float32),
        mesh=mesh,
        compiler_params=cp,
        scratch_types=[
            pltpu.VMEM((C,), jnp.int32),
            pltpu.VMEM((C,), jnp.int32),
            pltpu.VMEM((C,), jnp.int32),
            pltpu.VMEM((C,), jnp.int32),
            pltpu.VMEM((C,), jnp.int32),
            pltpu.VMEM((C, 16), jnp.float32),
            pltpu.VMEM((C, DIM), jnp.float32),
            pltpu.VMEM((C, DIM), jnp.float32),
            pltpu.VMEM((CH, CP), jnp.float32),
            pltpu.SemaphoreType.DMA,
        ],
    )
    def k(idx_hbm, e_hbm, p_hbm, out_hbm,
          idx_v, h0_v, h1_v, pr_v, col_v, pbuf, rows0, rows1, out_v, sem):
        wid = lax.convert_element_type(
            lax.axis_index("s") * NC + lax.axis_index("c"), jnp.int32
        )

        def chunk_body(cix):
            base = wid * _i32(PER_TILE) + cix * _i32(C)
            pltpu.sync_copy(idx_hbm.at[pl.ds(base, C)], idx_v)

            def hash_body(i):
                xv = idx_v[pl.ds(i, LANES)]
                h0_v[pl.ds(i, LANES)] = _hash_mod(xv, A0, B0)
                h1_v[pl.ds(i, LANES)] = _hash_mod(xv, A1, B1)
                pr_v[pl.ds(i, LANES)] = lax.shift_right_logical(xv, _i32(3))
                col_v[pl.ds(i, LANES)] = lax.shift_left(xv & _i32(7), _i32(1))

            _loop(C, LANES, hash_body)

            cps = []
            for g in range(C // GSUB):
                s = pl.ds(g * GSUB, GSUB)
                cps.append(pltpu.async_copy(e_hbm.at[h0_v.at[s]], rows0.at[s], sem))
                cps.append(pltpu.async_copy(e_hbm.at[h1_v.at[s]], rows1.at[s], sem))
                cps.append(pltpu.async_copy(p_hbm.at[pr_v.at[s]], pbuf.at[s], sem))
            for cp in cps:
                cp.wait()

            def extract_body(i):
                rowv = lax.iota(jnp.int32, 16) + i
                cv = col_v[pl.ds(i, LANES)]
                w0 = plsc.load_gather(pbuf, [rowv, cv])
                w1 = plsc.load_gather(pbuf, [rowv, cv + 1])
                out_v[DIM, pl.ds(i, LANES)] = w0
                out_v[DIM + 1, pl.ds(i, LANES)] = w1

            _loop(C, LANES, extract_body)

            r64 = jnp.full((LANES,), DIM, jnp.int32)
            r65 = jnp.full((LANES,), DIM + 1, jnp.int32)
            ciota = lax.iota(jnp.int32, 16)

            def combine_body(t):
                tt = jnp.full((LANES,), t, jnp.int32)
                w0 = plsc.load_gather(out_v, [r64, tt])
                w1 = plsc.load_gather(out_v, [r65, tt])
                for j in range(DIM // LANES):
                    sl = pl.ds(j * LANES, LANES)
                    v = rows0[t, sl] * w0 + rows1[t, sl] * w1
                    plsc.store_scatter(
                        out_v, [ciota + _i32(j * LANES), tt], v
                    )

            _loop(C, 1, combine_body)
            pltpu.sync_copy(
                out_v.at[:, pl.ds(0, C)], out_hbm.at[:, pl.ds(base, C)]
            )

        _loop(CHUNKS, 1, chunk_body)

    return k(idx, emb, p16)


def _tile_out(flat):
    wsub = 8
    blk = wsub * N_TOK

    def body(i_ref, o_ref):
        o_ref[...] = i_ref[...].reshape(1, wsub, N_TOK)

    return pl.pallas_call(
        body,
        grid=(CH, W_TOK // wsub),
        in_specs=[
            pl.BlockSpec(
                (blk,),
                lambda c, w: (c * np.int32(W_TOK // wsub) + w,),
            )
        ],
        out_specs=pl.BlockSpec(
            (1, wsub, N_TOK), lambda c, w: (c, w, np.int32(0))
        ),
        out_shape=jax.ShapeDtypeStruct((CH, W_TOK, N_TOK), jnp.float32),
    )(flat)


def kernel(indices, shared_embeddings, importance_weights):
    idx = indices.T.reshape(-1).astype(jnp.int32)
    e_lin = lax.dot_general(
        shared_embeddings.T,
        jnp.eye(DIM, dtype=jnp.float32),
        (((0,), (0,)), ((), ())),
    )
    p16 = importance_weights.reshape(NUM_EMB * K // 16, 16)
    emb_c = _sc_call(idx, e_lin, p16)
    out3 = _tile_out(emb_c.reshape(-1))
    return out3.transpose(2, 1, 0)


# speedup vs baseline: 1.1823x; 1.1823x over previous
import dataclasses

import jax
import jax.numpy as jnp
import numpy as np
from jax import lax
from jax.experimental import pallas as pl
from jax.experimental.pallas import tpu as pltpu
from jax.experimental.pallas import tpu_sc as plsc

NUM_EMB = 1000000
DIM = 64
K = 2
BINS = 99999
MOD = 2147483647
A0, B0 = 976369, 1014797
A1, B1 = 1982627, 180523
QMAX = (MOD - 1) // BINS

NC, NS, LANES = 2, 16, 16
NW = NC * NS

TOKENS = 4096 * 200
PER_TILE = TOKENS // NW
C = 256
CHUNKS = PER_TILE // C
GSUB = 128


def _i32(v):
    return jnp.int32(v)


def _loop(n, step, body):
    def bf(i, carry):
        body(i * _i32(step))
        return carry

    lax.fori_loop(_i32(0), _i32(n // step), bf, _i32(0))


def _hash_mod(xv, a, b):
    x0 = xv & _i32(1023)
    x1 = lax.shift_right_logical(xv, _i32(10))
    t_lo = x0 * _i32(a) + _i32(b)
    t_hi = x1 * _i32(a)
    u = lax.shift_right_logical(t_hi, _i32(21))
    v = t_hi & _i32((1 << 21) - 1)
    r1 = lax.shift_left(v, _i32(10)) + u
    s = r1 + t_lo
    s = (s & _i32(0x7FFFFFFF)) + lax.shift_right_logical(s, _i32(31))
    s = jnp.where(s >= _i32(MOD), s - _i32(MOD), s)
    q = (s.astype(jnp.float32) * jnp.float32(1.0 / BINS)).astype(jnp.int32)
    q = jnp.clip(q, _i32(0), _i32(QMAX))
    r = s - q * _i32(BINS)
    r = jnp.where(r < _i32(0), r + _i32(BINS), r)
    r = jnp.where(r >= _i32(BINS), r - _i32(BINS), r)
    return r


def _sc_call(idx, emb, p16):
    mesh = plsc.VectorSubcoreMesh(
        core_axis_name="c", subcore_axis_name="s", num_cores=NC, num_subcores=NS
    )

    cp = pltpu.CompilerParams()
    if "needs_layout_passes" in pltpu.CompilerParams.__dataclass_fields__:
        cp = dataclasses.replace(cp, needs_layout_passes=False)
    if "use_tc_tiling_on_sc" in pltpu.CompilerParams.__dataclass_fields__:
        cp = dataclasses.replace(cp, use_tc_tiling_on_sc=False)

    buf_set = [
        pltpu.VMEM((C,), jnp.int32),
        pltpu.VMEM((C,), jnp.int32),
        pltpu.VMEM((C,), jnp.int32),
        pltpu.VMEM((C,), jnp.int32),
        pltpu.VMEM((C,), jnp.int32),
        pltpu.VMEM((C, 16), jnp.float32),
        pltpu.VMEM((C, DIM), jnp.float32),
        pltpu.VMEM((C, DIM), jnp.float32),
        pltpu.VMEM((C, DIM + K), jnp.float32),
    ]

    @pl.kernel(
        out_type=jax.ShapeDtypeStruct((TOKENS, DIM + K), jnp.float32),
        mesh=mesh,
        compiler_params=cp,
        scratch_types=buf_set + buf_set + [
            pltpu.SemaphoreType.DMA,
            pltpu.SemaphoreType.DMA,
            pltpu.SemaphoreType.DMA,
            pltpu.SemaphoreType.DMA,
        ],
    )
    def k(idx_hbm, e_hbm, p_hbm, out_hbm, *bufs):
        A = bufs[0:9]
        B = bufs[9:18]
        semg_a, semg_b, semo_a, semo_b = bufs[18:22]

        wid = lax.convert_element_type(
            lax.axis_index("s") * NC + lax.axis_index("c"), jnp.int32
        )
        c64 = jnp.full((LANES,), DIM, jnp.int32)
        c65 = jnp.full((LANES,), DIM + 1, jnp.int32)

        def base_of(cix):
            return wid * _i32(PER_TILE) + cix * _i32(C)

        def prefetch(cix, bufset, semg):
            idx_v, h0_v, h1_v, pr_v, col_v, pbuf, rows0, rows1, _ = bufset
            pltpu.sync_copy(idx_hbm.at[pl.ds(base_of(cix), C)], idx_v)

            def hash_body(i):
                xv = idx_v[pl.ds(i, LANES)]
                h0_v[pl.ds(i, LANES)] = _hash_mod(xv, A0, B0)
                h1_v[pl.ds(i, LANES)] = _hash_mod(xv, A1, B1)
                pr_v[pl.ds(i, LANES)] = lax.shift_right_logical(xv, _i32(3))
                col_v[pl.ds(i, LANES)] = lax.shift_left(xv & _i32(7), _i32(1))

            _loop(C, LANES, hash_body)
            for g in range(C // GSUB):
                s = pl.ds(g * GSUB, GSUB)
                pltpu.async_copy(e_hbm.at[h0_v.at[s]], rows0.at[s], semg)
                pltpu.async_copy(e_hbm.at[h1_v.at[s]], rows1.at[s], semg)
                pltpu.async_copy(p_hbm.at[pr_v.at[s]], pbuf.at[s], semg)

        def drain_gathers(bufset, semg):
            _, _, _, _, _, pbuf, rows0, rows1, _ = bufset
            for g in range(C // GSUB):
                s = pl.ds(g * GSUB, GSUB)
                hs = pl.ds(0, GSUB)
                pltpu.make_async_copy(e_hbm.at[hs], rows0.at[s], semg).wait()
                pltpu.make_async_copy(e_hbm.at[hs], rows1.at[s], semg).wait()
                pltpu.make_async_copy(p_hbm.at[hs], pbuf.at[s], semg).wait()

        def compute(cix, bufset, semo, drain_out):
            _, _, _, _, col_v, pbuf, rows0, rows1, out_v = bufset

            @pl.when(drain_out)
            def _():
                pltpu.make_async_copy(
                    out_v, out_hbm.at[pl.ds(0, C)], semo
                ).wait()

            def extract_body(i):
                rowv = lax.iota(jnp.int32, 16) + i
                cv = col_v[pl.ds(i, LANES)]
                w0 = plsc.load_gather(pbuf, [rowv, cv])
                w1 = plsc.load_gather(pbuf, [rowv, cv + 1])
                plsc.store_scatter(out_v, [rowv, c64], w0)
                plsc.store_scatter(out_v, [rowv, c65], w1)

            _loop(C, LANES, extract_body)

            def combine_body(t):
                tt = jnp.full((LANES,), t, jnp.int32)
                w0 = plsc.load_gather(out_v, [tt, c64])
                w1 = plsc.load_gather(out_v, [tt, c65])
                for j in range(DIM // LANES):
                    sl = pl.ds(j * LANES, LANES)
                    out_v[t, sl] = rows0[t, sl] * w0 + rows1[t, sl] * w1

            _loop(C, 1, combine_body)
            pltpu.async_copy(out_v, out_hbm.at[pl.ds(base_of(cix), C)], semo)

        prefetch(_i32(0), A, semg_a)

        def pair_body(c):
            prefetch(c + _i32(1), B, semg_b)
            drain_gathers(A, semg_a)
            compute(c, A, semo_a, c >= _i32(2))

            @pl.when(c + _i32(2) < _i32(CHUNKS))
            def _():
                prefetch(c + _i32(2), A, semg_a)

            drain_gathers(B, semg_b)
            compute(c + _i32(1), B, semo_b, c >= _i32(2))

        _loop(CHUNKS, 2, pair_body)
        pltpu.make_async_copy(A[8], out_hbm.at[pl.ds(0, C)], semo_a).wait()
        pltpu.make_async_copy(B[8], out_hbm.at[pl.ds(0, C)], semo_b).wait()

    return k(idx, emb, p16)


def kernel(indices, shared_embeddings, importance_weights):
    n, w = indices.shape
    idx = indices.reshape(-1).astype(jnp.int32)
    e_lin = lax.dot_general(
        shared_embeddings.T,
        jnp.eye(DIM, dtype=jnp.float32),
        (((0,), (0,)), ((), ())),
    )
    p16 = importance_weights.reshape(NUM_EMB * K // 16, 16)
    out = _sc_call(idx, e_lin, p16)
    return out.reshape(n, w, DIM + K)


# speedup vs baseline: 1.1938x; 1.0098x over previous
import dataclasses

import jax
import jax.numpy as jnp
import numpy as np
from jax import lax
from jax.experimental import pallas as pl
from jax.experimental.pallas import tpu as pltpu
from jax.experimental.pallas import tpu_sc as plsc

NUM_EMB = 1000000
DIM = 64
K = 2
BINS = 99999
MOD = 2147483647
A0, B0 = 976369, 1014797
A1, B1 = 1982627, 180523
QMAX = (MOD - 1) // BINS

NC, NS, LANES = 2, 16, 16
NW = NC * NS

TOKENS = 4096 * 200
PER_TILE = TOKENS // NW
C = 256
CHUNKS = PER_TILE // C
GSUB = 128


def _i32(v):
    return jnp.int32(v)


def _loop(n, step, body, unroll=1):
    def bf(i, carry):
        b = i * _i32(step * unroll)
        for u in range(unroll):
            body(b + _i32(u * step))
        return carry

    lax.fori_loop(_i32(0), _i32(n // (step * unroll)), bf, _i32(0))


def _hash_mod(xv, a, b):
    x0 = xv & _i32(1023)
    x1 = lax.shift_right_logical(xv, _i32(10))
    t_lo = x0 * _i32(a) + _i32(b)
    t_hi = x1 * _i32(a)
    u = lax.shift_right_logical(t_hi, _i32(21))
    v = t_hi & _i32((1 << 21) - 1)
    r1 = lax.shift_left(v, _i32(10)) + u
    s = r1 + t_lo
    s = (s & _i32(0x7FFFFFFF)) + lax.shift_right_logical(s, _i32(31))
    s = jnp.where(s >= _i32(MOD), s - _i32(MOD), s)
    q = (s.astype(jnp.float32) * jnp.float32(1.0 / BINS)).astype(jnp.int32)
    q = jnp.clip(q, _i32(0), _i32(QMAX))
    r = s - q * _i32(BINS)
    r = jnp.where(r < _i32(0), r + _i32(BINS), r)
    r = jnp.where(r >= _i32(BINS), r - _i32(BINS), r)
    return r


def _sc_call(idx, emb, p16):
    mesh = plsc.VectorSubcoreMesh(
        core_axis_name="c", subcore_axis_name="s", num_cores=NC, num_subcores=NS
    )

    cp = pltpu.CompilerParams()
    if "needs_layout_passes" in pltpu.CompilerParams.__dataclass_fields__:
        cp = dataclasses.replace(cp, needs_layout_passes=False)
    if "use_tc_tiling_on_sc" in pltpu.CompilerParams.__dataclass_fields__:
        cp = dataclasses.replace(cp, use_tc_tiling_on_sc=False)

    buf_set = [
        pltpu.VMEM((C,), jnp.int32),
        pltpu.VMEM((C,), jnp.int32),
        pltpu.VMEM((C,), jnp.int32),
        pltpu.VMEM((C,), jnp.int32),
        pltpu.VMEM((C,), jnp.int32),
        pltpu.VMEM((C, 16), jnp.float32),
        pltpu.VMEM((C, DIM), jnp.float32),
        pltpu.VMEM((C, DIM), jnp.float32),
        pltpu.VMEM((C, DIM + K), jnp.float32),
    ]

    @pl.kernel(
        out_type=jax.ShapeDtypeStruct((TOKENS, DIM + K), jnp.float32),
        mesh=mesh,
        compiler_params=cp,
        scratch_types=buf_set + buf_set + [
            pltpu.SemaphoreType.DMA,
            pltpu.SemaphoreType.DMA,
            pltpu.SemaphoreType.DMA,
            pltpu.SemaphoreType.DMA,
        ],
    )
    def k(idx_hbm, e_hbm, p_hbm, out_hbm, *bufs):
        A = bufs[0:9]
        B = bufs[9:18]
        semg_a, semg_b, semo_a, semo_b = bufs[18:22]

        wid = lax.convert_element_type(
            lax.axis_index("s") * NC + lax.axis_index("c"), jnp.int32
        )
        c64 = jnp.full((LANES,), DIM, jnp.int32)
        c65 = jnp.full((LANES,), DIM + 1, jnp.int32)

        def base_of(cix):
            return wid * _i32(PER_TILE) + cix * _i32(C)

        def prefetch(cix, bufset, semg):
            idx_v, h0_v, h1_v, pr_v, col_v, pbuf, rows0, rows1, _ = bufset
            pltpu.sync_copy(idx_hbm.at[pl.ds(base_of(cix), C)], idx_v)

            def hash_body(i):
                xv = idx_v[pl.ds(i, LANES)]
                h0_v[pl.ds(i, LANES)] = _hash_mod(xv, A0, B0)
                h1_v[pl.ds(i, LANES)] = _hash_mod(xv, A1, B1)
                pr_v[pl.ds(i, LANES)] = lax.shift_right_logical(xv, _i32(3))
                col_v[pl.ds(i, LANES)] = lax.shift_left(xv & _i32(7), _i32(1))

            _loop(C, LANES, hash_body)
            for g in range(C // GSUB):
                s = pl.ds(g * GSUB, GSUB)
                pltpu.async_copy(e_hbm.at[h0_v.at[s]], rows0.at[s], semg)
                pltpu.async_copy(e_hbm.at[h1_v.at[s]], rows1.at[s], semg)
                pltpu.async_copy(p_hbm.at[pr_v.at[s]], pbuf.at[s], semg)

        def drain_gathers(bufset, semg):
            _, _, _, _, _, pbuf, rows0, rows1, _ = bufset
            for g in range(C // GSUB):
                s = pl.ds(g * GSUB, GSUB)
                hs = pl.ds(0, GSUB)
                pltpu.make_async_copy(e_hbm.at[hs], rows0.at[s], semg).wait()
                pltpu.make_async_copy(e_hbm.at[hs], rows1.at[s], semg).wait()
                pltpu.make_async_copy(p_hbm.at[hs], pbuf.at[s], semg).wait()

        def compute(cix, bufset, semo, drain_out):
            _, _, _, _, col_v, pbuf, rows0, rows1, out_v = bufset

            @pl.when(drain_out)
            def _():
                pltpu.make_async_copy(
                    out_v, out_hbm.at[pl.ds(0, C)], semo
                ).wait()

            def extract_body(i):
                rowv = lax.iota(jnp.int32, 16) + i
                cv = col_v[pl.ds(i, LANES)]
                w0 = plsc.load_gather(pbuf, [rowv, cv])
                w1 = plsc.load_gather(pbuf, [rowv, cv + 1])
                plsc.store_scatter(out_v, [rowv, c64], w0)
                plsc.store_scatter(out_v, [rowv, c65], w1)

            _loop(C, LANES, extract_body)

            def combine_body(t):
                tt = jnp.full((LANES,), t, jnp.int32)
                w0 = plsc.load_gather(out_v, [tt, c64])
                w1 = plsc.load_gather(out_v, [tt, c65])
                for j in range(DIM // LANES):
                    sl = pl.ds(j * LANES, LANES)
                    out_v[t, sl] = rows0[t, sl] * w0 + rows1[t, sl] * w1

            _loop(C, 1, combine_body, unroll=4)
            pltpu.async_copy(out_v, out_hbm.at[pl.ds(base_of(cix), C)], semo)

        prefetch(_i32(0), A, semg_a)

        def pair_body(c):
            prefetch(c + _i32(1), B, semg_b)
            drain_gathers(A, semg_a)
            compute(c, A, semo_a, c >= _i32(2))

            @pl.when(c + _i32(2) < _i32(CHUNKS))
            def _():
                prefetch(c + _i32(2), A, semg_a)

            drain_gathers(B, semg_b)
            compute(c + _i32(1), B, semo_b, c >= _i32(2))

        _loop(CHUNKS, 2, pair_body)
        pltpu.make_async_copy(A[8], out_hbm.at[pl.ds(0, C)], semo_a).wait()
        pltpu.make_async_copy(B[8], out_hbm.at[pl.ds(0, C)], semo_b).wait()

    return k(idx, emb, p16)


def kernel(indices, shared_embeddings, importance_weights):
    n, w = indices.shape
    idx = indices.reshape(-1).astype(jnp.int32)
    e_lin = lax.dot_general(
        shared_embeddings.T,
        jnp.eye(DIM, dtype=jnp.float32),
        (((0,), (0,)), ((), ())),
    )
    p16 = importance_weights.reshape(NUM_EMB * K // 16, 16)
    out = _sc_call(idx, e_lin, p16)
    return out.reshape(n, w, DIM + K)


# speedup vs baseline: 1.1962x; 1.0020x over previous
import dataclasses

import jax
import jax.numpy as jnp
import numpy as np
from jax import lax
from jax.experimental import pallas as pl
from jax.experimental.pallas import tpu as pltpu
from jax.experimental.pallas import tpu_sc as plsc

NUM_EMB = 1000000
DIM = 64
K = 2
BINS = 99999
MOD = 2147483647
A0, B0 = 976369, 1014797
A1, B1 = 1982627, 180523
QMAX = (MOD - 1) // BINS

NC, NS, LANES = 2, 16, 16
NW = NC * NS

TOKENS = 4096 * 200
PER_TILE = TOKENS // NW
C = 256
CHUNKS = PER_TILE // C
GSUB = 128


def _i32(v):
    return jnp.int32(v)


def _loop(n, step, body, unroll=1):
    def bf(i, carry):
        b = i * _i32(step * unroll)
        for u in range(unroll):
            body(b + _i32(u * step))
        return carry

    lax.fori_loop(_i32(0), _i32(n // (step * unroll)), bf, _i32(0))


def _hash_mod(xv, a, b):
    x0 = xv & _i32(1023)
    x1 = lax.shift_right_logical(xv, _i32(10))
    t_lo = x0 * _i32(a) + _i32(b)
    t_hi = x1 * _i32(a)
    u = lax.shift_right_logical(t_hi, _i32(21))
    v = t_hi & _i32((1 << 21) - 1)
    r1 = lax.shift_left(v, _i32(10)) + u
    s = r1 + t_lo
    s = (s & _i32(0x7FFFFFFF)) + lax.shift_right_logical(s, _i32(31))
    s = jnp.where(s >= _i32(MOD), s - _i32(MOD), s)
    q = (s.astype(jnp.float32) * jnp.float32(1.0 / BINS)).astype(jnp.int32)
    q = jnp.clip(q, _i32(0), _i32(QMAX))
    r = s - q * _i32(BINS)
    r = jnp.where(r < _i32(0), r + _i32(BINS), r)
    r = jnp.where(r >= _i32(BINS), r - _i32(BINS), r)
    return r


def _sc_call(idx, emb, p16):
    mesh = plsc.VectorSubcoreMesh(
        core_axis_name="c", subcore_axis_name="s", num_cores=NC, num_subcores=NS
    )

    cp = pltpu.CompilerParams()
    if "needs_layout_passes" in pltpu.CompilerParams.__dataclass_fields__:
        cp = dataclasses.replace(cp, needs_layout_passes=False)
    if "use_tc_tiling_on_sc" in pltpu.CompilerParams.__dataclass_fields__:
        cp = dataclasses.replace(cp, use_tc_tiling_on_sc=False)

    buf_set = [
        pltpu.VMEM((C,), jnp.int32),
        pltpu.VMEM((C,), jnp.int32),
        pltpu.VMEM((C,), jnp.int32),
        pltpu.VMEM((C,), jnp.int32),
        pltpu.VMEM((C,), jnp.int32),
        pltpu.VMEM((C, 16), jnp.float32),
        pltpu.VMEM((C, DIM), jnp.float32),
        pltpu.VMEM((C, DIM), jnp.float32),
        pltpu.VMEM((C, DIM + K), jnp.float32),
    ]

    @pl.kernel(
        out_type=jax.ShapeDtypeStruct((TOKENS, DIM + K), jnp.float32),
        mesh=mesh,
        compiler_params=cp,
        scratch_types=buf_set + buf_set + [
            pltpu.SemaphoreType.DMA,
            pltpu.SemaphoreType.DMA,
            pltpu.SemaphoreType.DMA,
            pltpu.SemaphoreType.DMA,
        ],
    )
    def k(idx_hbm, e_hbm, p_hbm, out_hbm, *bufs):
        A = bufs[0:9]
        B = bufs[9:18]
        semg_a, semg_b, semo_a, semo_b = bufs[18:22]

        wid = lax.convert_element_type(
            lax.axis_index("s") * NC + lax.axis_index("c"), jnp.int32
        )
        c64 = jnp.full((LANES,), DIM, jnp.int32)
        c65 = jnp.full((LANES,), DIM + 1, jnp.int32)

        def base_of(cix):
            return wid * _i32(PER_TILE) + cix * _i32(C)

        def prefetch(cix, bufset, semg):
            idx_v, h0_v, h1_v, pr_v, col_v, pbuf, rows0, rows1, _ = bufset
            pltpu.sync_copy(idx_hbm.at[pl.ds(base_of(cix), C)], idx_v)

            def hash_body(i):
                xv = idx_v[pl.ds(i, LANES)]
                h0_v[pl.ds(i, LANES)] = _hash_mod(xv, A0, B0)
                h1_v[pl.ds(i, LANES)] = _hash_mod(xv, A1, B1)
                pr_v[pl.ds(i, LANES)] = lax.shift_right_logical(xv, _i32(3))
                col_v[pl.ds(i, LANES)] = lax.shift_left(xv & _i32(7), _i32(1))

            _loop(C, LANES, hash_body, unroll=2)
            for g in range(C // GSUB):
                s = pl.ds(g * GSUB, GSUB)
                pltpu.async_copy(e_hbm.at[h0_v.at[s]], rows0.at[s], semg)
                pltpu.async_copy(e_hbm.at[h1_v.at[s]], rows1.at[s], semg)
                pltpu.async_copy(p_hbm.at[pr_v.at[s]], pbuf.at[s], semg)

        def drain_gathers(bufset, semg):
            _, _, _, _, _, pbuf, rows0, rows1, _ = bufset
            for g in range(C // GSUB):
                s = pl.ds(g * GSUB, GSUB)
                hs = pl.ds(0, GSUB)
                pltpu.make_async_copy(e_hbm.at[hs], rows0.at[s], semg).wait()
                pltpu.make_async_copy(e_hbm.at[hs], rows1.at[s], semg).wait()
                pltpu.make_async_copy(p_hbm.at[hs], pbuf.at[s], semg).wait()

        def compute(cix, bufset, semo, drain_out):
            _, _, _, _, col_v, pbuf, rows0, rows1, out_v = bufset

            @pl.when(drain_out)
            def _():
                pltpu.make_async_copy(
                    out_v, out_hbm.at[pl.ds(0, C)], semo
                ).wait()

            def extract_body(i):
                rowv = lax.iota(jnp.int32, 16) + i
                cv = col_v[pl.ds(i, LANES)]
                w0 = plsc.load_gather(pbuf, [rowv, cv])
                w1 = plsc.load_gather(pbuf, [rowv, cv + 1])
                plsc.store_scatter(out_v, [rowv, c64], w0)
                plsc.store_scatter(out_v, [rowv, c65], w1)

            _loop(C, LANES, extract_body, unroll=2)

            def combine_body(t):
                tt = jnp.full((LANES,), t, jnp.int32)
                w0 = plsc.load_gather(out_v, [tt, c64])
                w1 = plsc.load_gather(out_v, [tt, c65])
                for j in range(DIM // LANES):
                    sl = pl.ds(j * LANES, LANES)
                    out_v[t, sl] = rows0[t, sl] * w0 + rows1[t, sl] * w1

            _loop(C, 1, combine_body, unroll=4)
            pltpu.async_copy(out_v, out_hbm.at[pl.ds(base_of(cix), C)], semo)

        prefetch(_i32(0), A, semg_a)

        def pair_body(c):
            prefetch(c + _i32(1), B, semg_b)
            drain_gathers(A, semg_a)
            compute(c, A, semo_a, c >= _i32(2))

            @pl.when(c + _i32(2) < _i32(CHUNKS))
            def _():
                prefetch(c + _i32(2), A, semg_a)

            drain_gathers(B, semg_b)
            compute(c + _i32(1), B, semo_b, c >= _i32(2))

        _loop(CHUNKS, 2, pair_body)
        pltpu.make_async_copy(A[8], out_hbm.at[pl.ds(0, C)], semo_a).wait()
        pltpu.make_async_copy(B[8], out_hbm.at[pl.ds(0, C)], semo_b).wait()

    return k(idx, emb, p16)


def kernel(indices, shared_embeddings, importance_weights):
    n, w = indices.shape
    idx = indices.reshape(-1).astype(jnp.int32)
    e_lin = lax.dot_general(
        shared_embeddings.T,
        jnp.eye(DIM, dtype=jnp.float32),
        (((0,), (0,)), ((), ())),
    )
    p16 = importance_weights.reshape(NUM_EMB * K // 16, 16)
    out = _sc_call(idx, e_lin, p16)
    return out.reshape(n, w, DIM + K)
